# Initial kernel scaffold; baseline (speedup 1.0000x reference)
import functools

import jax
import jax.numpy as jnp
from jax import lax
from jax.experimental import pallas as pl
from jax.experimental.pallas import tpu as pltpu
from jax.experimental.pallas import tpu_sc as plsc

_VOCAB = 100002
_HIDDEN = 32
_SAMPLES = 20
_BATCH = 16384
_TOTAL = _BATCH * _SAMPLES
_NC, _NS = 2, 16
_NW = _NC * _NS
_ROWS_PER_W = _TOTAL // _NW
_G = 128
_MACRO = 2048
_NSTREAM = _MACRO // _G
_NMACRO = _ROWS_PER_W // _MACRO
_IDX_ROWS_PER_MACRO = _MACRO // _G

_sc_mesh = plsc.VectorSubcoreMesh(core_axis_name="c", subcore_axis_name="s")


@functools.partial(
    pl.kernel,
    mesh=_sc_mesh,
    out_type=jax.ShapeDtypeStruct((_TOTAL, _HIDDEN), jnp.float32),
    scratch_types=[
        pltpu.VMEM((_NSTREAM, _G), jnp.int32),
        pltpu.VMEM((_MACRO, _HIDDEN), jnp.float32),
        pltpu.SemaphoreType.DMA,
    ],
)
def _gather_sc(idx_hbm, table_hbm, out_hbm, idx_v, rows_v, sem):
    wid = lax.axis_index("s") * _NC + lax.axis_index("c")
    row_base = wid * _ROWS_PER_W
    idx_base = wid * (_ROWS_PER_W // _G)

    def body(m, carry):
        mrow = row_base + m * _MACRO
        midx = idx_base + m * _IDX_ROWS_PER_MACRO
        pltpu.sync_copy(idx_hbm.at[pl.ds(midx, _IDX_ROWS_PER_MACRO)], idx_v)
        copies = []
        for j in range(_NSTREAM):
            copies.append(
                pltpu.async_copy(
                    table_hbm.at[idx_v.at[j]],
                    rows_v.at[pl.ds(j * _G, _G)],
                    sem,
                )
            )
        for c in copies:
            c.wait()
        pltpu.sync_copy(rows_v, out_hbm.at[pl.ds(mrow, _MACRO)])
        return carry

    lax.fori_loop(0, _NMACRO, body, 0)


def _mm_body(x_ref, w_ref, b_ref, o_ref):
    o_ref[...] = (
        lax.dot_general(
            x_ref[...], w_ref[...],
            (((1,), (1,)), ((), ())),
            preferred_element_type=jnp.float32,
        )
        + b_ref[...]
    )


_BM = 2048


def _project_tc(emb2d, W, b2d):
    return pl.pallas_call(
        _mm_body,
        grid=(_BATCH // _BM,),
        in_specs=[
            pl.BlockSpec((_BM, _SAMPLES * _HIDDEN), lambda i: (i, 0)),
            pl.BlockSpec((_HIDDEN, _SAMPLES * _HIDDEN), lambda i: (0, 0)),
            pl.BlockSpec((1, _HIDDEN), lambda i: (0, 0)),
        ],
        out_specs=pl.BlockSpec((_BM, _HIDDEN), lambda i: (i, 0)),
        out_shape=jax.ShapeDtypeStruct((_BATCH, _HIDDEN), jnp.float32),
    )(emb2d, W, b2d)


def kernel(all_values, table, W, b):
    idx2d = all_values.reshape(_TOTAL // _G, _G).astype(jnp.int32)
    emb = _gather_sc(idx2d, table)
    emb2d = emb.reshape(_BATCH, _SAMPLES * _HIDDEN)
    return _project_tc(emb2d, W, b.reshape(1, _HIDDEN))


# speedup vs baseline: 9.8945x; 9.8945x over previous
import functools

import jax
import jax.numpy as jnp
from jax import lax
from jax.experimental import pallas as pl
from jax.experimental.pallas import tpu as pltpu
from jax.experimental.pallas import tpu_sc as plsc

_VOCAB = 100002
_HIDDEN = 32
_SAMPLES = 20
_BATCH = 16384
_TOTAL = _BATCH * _SAMPLES
_NC, _NS = 2, 16
_NW = _NC * _NS
_ROWS_PER_W = _TOTAL // _NW
_G = 128
_MACRO = 2048
_NSTREAM = _MACRO // _G
_NMACRO = _ROWS_PER_W // _MACRO
_IDX_ROWS_PER_MACRO = _MACRO // _G

_sc_mesh = plsc.VectorSubcoreMesh(core_axis_name="c", subcore_axis_name="s")


@functools.partial(
    pl.kernel,
    mesh=_sc_mesh,
    out_type=jax.ShapeDtypeStruct((_TOTAL, _HIDDEN), jnp.float32),
    scratch_types=[
        pltpu.VMEM((_NSTREAM, _G), jnp.int32),
        pltpu.VMEM((_MACRO, _HIDDEN), jnp.float32),
        pltpu.SemaphoreType.DMA,
    ],
    compiler_params=pltpu.CompilerParams(use_tc_tiling_on_sc=False),
)
def _gather_sc(idx_hbm, table_hbm, out_hbm, idx_v, rows_v, sem):
    wid = lax.axis_index("s") * _NC + lax.axis_index("c")
    row_base = wid * _ROWS_PER_W
    idx_base = wid * (_ROWS_PER_W // _G)

    def body(m, carry):
        mrow = row_base + m * _MACRO
        midx = idx_base + m * _IDX_ROWS_PER_MACRO
        pltpu.sync_copy(idx_hbm.at[pl.ds(midx, _IDX_ROWS_PER_MACRO)], idx_v)
        copies = []
        for j in range(_NSTREAM):
            copies.append(
                pltpu.async_copy(
                    table_hbm.at[idx_v.at[j]],
                    rows_v.at[pl.ds(j * _G, _G)],
                    sem,
                )
            )
        for c in copies:
            c.wait()
        pltpu.sync_copy(rows_v, out_hbm.at[pl.ds(mrow, _MACRO)])
        return carry

    lax.fori_loop(0, _NMACRO, body, 0)


def _mm_body(x_ref, w_ref, b_ref, o_ref):
    o_ref[...] = (
        lax.dot_general(
            x_ref[...], w_ref[...],
            (((1,), (1,)), ((), ())),
            preferred_element_type=jnp.float32,
        )
        + b_ref[...]
    )


_BM = 2048


def _project_tc(emb2d, W, b2d):
    return pl.pallas_call(
        _mm_body,
        grid=(_BATCH // _BM,),
        in_specs=[
            pl.BlockSpec((_BM, _SAMPLES * _HIDDEN), lambda i: (i, 0)),
            pl.BlockSpec((_HIDDEN, _SAMPLES * _HIDDEN), lambda i: (0, 0)),
            pl.BlockSpec((1, _HIDDEN), lambda i: (0, 0)),
        ],
        out_specs=pl.BlockSpec((_BM, _HIDDEN), lambda i: (i, 0)),
        out_shape=jax.ShapeDtypeStruct((_BATCH, _HIDDEN), jnp.float32),
    )(emb2d, W, b2d)


def kernel(all_values, table, W, b):
    idx2d = all_values.reshape(_TOTAL // _G, _G).astype(jnp.int32)
    emb = _gather_sc(idx2d, table)
    emb2d = emb.reshape(_BATCH, _SAMPLES * _HIDDEN)
    return _project_tc(emb2d, W, b.reshape(1, _HIDDEN))


# speedup vs baseline: 10.8194x; 1.0935x over previous
import functools

import jax
import jax.numpy as jnp
from jax import lax
from jax.experimental import pallas as pl
from jax.experimental.pallas import tpu as pltpu
from jax.experimental.pallas import tpu_sc as plsc

_VOCAB = 100002
_HIDDEN = 32
_SAMPLES = 20
_BATCH = 16384
_TOTAL = _BATCH * _SAMPLES
_NC, _NS = 2, 16
_NW = _NC * _NS
_IPW = _BATCH // _NW
_G = 128
_MI = 128
_MROWS = _MI * _SAMPLES
_NSTREAM = _MROWS // _G
_NMACRO = _IPW // _MI
_NSLAB = 5
_SLAB = _TOTAL // _NSLAB

_sc_mesh = plsc.VectorSubcoreMesh(core_axis_name="c", subcore_axis_name="s")


@functools.partial(
    pl.kernel,
    mesh=_sc_mesh,
    out_type=jax.ShapeDtypeStruct((_TOTAL, _HIDDEN), jnp.float32),
    scratch_types=[
        pltpu.VMEM((_NSTREAM, _G), jnp.int32),
        pltpu.VMEM((_NSTREAM, _G), jnp.int32),
        pltpu.VMEM((_MROWS, _HIDDEN), jnp.float32),
        pltpu.SemaphoreType.DMA,
    ],
    compiler_params=pltpu.CompilerParams(
        use_tc_tiling_on_sc=False, needs_layout_passes=False
    ),
)
def _gather_sc(idx_hbm, table_hbm, out_hbm, idx_v, idxp_v, rows_v, sem):
    wid = lax.axis_index("s") * _NC + lax.axis_index("c")
    i0 = wid * _IPW
    idx_row0 = (i0 * _SAMPLES) // _G

    def body(m, carry):
        pltpu.sync_copy(
            idx_hbm.at[pl.ds(idx_row0 + m * _NSTREAM, _NSTREAM)], idx_v
        )
        for k in range(_NSTREAM):
            for lg in range(_G // 16):
                pp = k * _G + lg * 16 + lax.iota(jnp.int32, 16)
                c = lax.shift_right_logical(pp, 9)
                j = lax.bitwise_and(pp, jnp.int32(511))
                p = (
                    lax.shift_right_logical(j, 2) * _SAMPLES
                    + c * 4
                    + lax.bitwise_and(j, jnp.int32(3))
                )
                vals = plsc.load_gather(
                    idx_v,
                    [lax.shift_right_logical(p, 7),
                     lax.bitwise_and(p, jnp.int32(127))],
                )
                idxp_v[k, pl.ds(lg * 16, 16)] = vals
        copies = []
        for k in range(_NSTREAM):
            copies.append(
                pltpu.async_copy(
                    table_hbm.at[idxp_v.at[k]],
                    rows_v.at[pl.ds(k * _G, _G)],
                    sem,
                )
            )
        for c in copies:
            c.wait()
        for c in range(_NSLAB):
            pltpu.sync_copy(
                rows_v.at[pl.ds(c * (_MI * 4), _MI * 4)],
                out_hbm.at[pl.ds(c * _SLAB + (i0 + m * _MI) * 4, _MI * 4)],
            )
        return carry

    lax.fori_loop(0, _NMACRO, body, 0)


def _mm_body(x_ref, w_ref, b_ref, o_ref):
    c = pl.program_id(1)
    acc = lax.dot_general(
        x_ref[0], w_ref[...],
        (((1,), (1,)), ((), ())),
        preferred_element_type=jnp.float32,
    )

    @pl.when(c == 0)
    def _():
        o_ref[...] = acc + b_ref[...]

    @pl.when(c != 0)
    def _():
        o_ref[...] += acc


_BM = 2048


def _project_tc(emb3, W, b2d):
    return pl.pallas_call(
        _mm_body,
        grid=(_BATCH // _BM, _NSLAB),
        in_specs=[
            pl.BlockSpec((1, _BM, 128), lambda i, c: (c, i, 0)),
            pl.BlockSpec((_HIDDEN, 128), lambda i, c: (0, c)),
            pl.BlockSpec((1, _HIDDEN), lambda i, c: (0, 0)),
        ],
        out_specs=pl.BlockSpec((_BM, _HIDDEN), lambda i, c: (i, 0)),
        out_shape=jax.ShapeDtypeStruct((_BATCH, _HIDDEN), jnp.float32),
    )(emb3, W, b2d)


def kernel(all_values, table, W, b):
    idx2d = all_values.reshape(_TOTAL // _G, _G)
    emb = _gather_sc(idx2d, table)
    emb3 = emb.reshape(_NSLAB, _BATCH, 4 * _HIDDEN)
    return _project_tc(emb3, W, b.reshape(1, _HIDDEN))


# speedup vs baseline: 12.2963x; 1.1365x over previous
import functools

import jax
import jax.numpy as jnp
from jax import lax
from jax.experimental import pallas as pl
from jax.experimental.pallas import tpu as pltpu
from jax.experimental.pallas import tpu_sc as plsc

_VOCAB = 100002
_HIDDEN = 32
_SAMPLES = 20
_BATCH = 16384
_TOTAL = _BATCH * _SAMPLES
_NC, _NS = 2, 16
_NW = _NC * _NS
_IPW = _BATCH // _NW
_G = 128
_MI = 128
_MROWS = _MI * _SAMPLES
_NSTREAM = _MROWS // _G
_NMACRO = _IPW // _MI
_NSLAB = 5
_SLAB = _TOTAL // _NSLAB

_sc_mesh = plsc.VectorSubcoreMesh(core_axis_name="c", subcore_axis_name="s")


@functools.partial(
    pl.kernel,
    mesh=_sc_mesh,
    out_type=jax.ShapeDtypeStruct((_TOTAL, _HIDDEN), jnp.float32),
    scratch_types=[
        pltpu.VMEM((_NSTREAM, _G), jnp.int32),
        pltpu.VMEM((_NSTREAM, _G), jnp.int32),
        pltpu.VMEM((_MROWS, _HIDDEN), jnp.float32),
        pltpu.SemaphoreType.DMA,
    ],
    compiler_params=pltpu.CompilerParams(
        use_tc_tiling_on_sc=False, needs_layout_passes=False
    ),
)
def _gather_sc(idx_hbm, table_hbm, out_hbm, idx_v, idxp_v, rows_v, sem):
    wid = lax.axis_index("s") * _NC + lax.axis_index("c")
    i0 = wid * _IPW
    idx_row0 = (i0 * _SAMPLES) // _G

    def body(m, carry):
        pltpu.sync_copy(
            idx_hbm.at[pl.ds(idx_row0 + m * _NSTREAM, _NSTREAM)], idx_v
        )
        for k in range(_NSTREAM):
            for lg in range(_G // 16):
                pp = k * _G + lg * 16 + lax.iota(jnp.int32, 16)
                c = lax.shift_right_logical(pp, 9)
                j = lax.bitwise_and(pp, jnp.int32(511))
                p = (
                    lax.shift_right_logical(j, 2) * _SAMPLES
                    + c * 4
                    + lax.bitwise_and(j, jnp.int32(3))
                )
                vals = plsc.load_gather(
                    idx_v,
                    [lax.shift_right_logical(p, 7),
                     lax.bitwise_and(p, jnp.int32(127))],
                )
                idxp_v[k, pl.ds(lg * 16, 16)] = vals
        copies = []
        for k in range(_NSTREAM):
            copies.append(
                pltpu.async_copy(
                    table_hbm.at[idxp_v.at[k]],
                    rows_v.at[pl.ds(k * _G, _G)],
                    sem,
                )
            )
        for c in copies:
            c.wait()
        for c in range(_NSLAB):
            pltpu.sync_copy(
                rows_v.at[pl.ds(c * (_MI * 4), _MI * 4)],
                out_hbm.at[pl.ds(c * _SLAB + (i0 + m * _MI) * 4, _MI * 4)],
            )
        return carry

    lax.fori_loop(0, _NMACRO, body, 0)


def _mm_body(x_ref, w_ref, b_ref, o_ref):
    acc = b_ref[...].astype(jnp.float32)
    for c in range(_NSLAB):
        acc = acc + lax.dot_general(
            x_ref[c], w_ref[:, c * 128:(c + 1) * 128],
            (((1,), (1,)), ((), ())),
            preferred_element_type=jnp.float32,
        )
    o_ref[...] = acc


_BM = 2048


def _project_tc(emb3, W, b2d):
    return pl.pallas_call(
        _mm_body,
        grid=(_BATCH // _BM,),
        in_specs=[
            pl.BlockSpec((_NSLAB, _BM, 128), lambda i: (0, i, 0)),
            pl.BlockSpec((_HIDDEN, _NSLAB * 128), lambda i: (0, 0)),
            pl.BlockSpec((1, _HIDDEN), lambda i: (0, 0)),
        ],
        out_specs=pl.BlockSpec((_BM, _HIDDEN), lambda i: (i, 0)),
        out_shape=jax.ShapeDtypeStruct((_BATCH, _HIDDEN), jnp.float32),
    )(emb3, W, b2d)


def kernel(all_values, table, W, b):
    idx2d = all_values.reshape(_TOTAL // _G, _G)
    emb = _gather_sc(idx2d, table)
    emb3 = emb.reshape(_NSLAB, _BATCH, 4 * _HIDDEN)
    return _project_tc(emb3, W, b.reshape(1, _HIDDEN))


# speedup vs baseline: 13.3332x; 1.0843x over previous
import functools

import jax
import jax.numpy as jnp
from jax import lax
from jax.experimental import pallas as pl
from jax.experimental.pallas import tpu as pltpu
from jax.experimental.pallas import tpu_sc as plsc

_VOCAB = 100002
_HIDDEN = 32
_SAMPLES = 20
_BATCH = 16384
_NC, _NS = 2, 16
_NW = _NC * _NS
_IPW = _BATCH // _NW
_MI = 128
_NMACRO = _IPW // _MI
_PACK = 128 // _HIDDEN

_sc_mesh = plsc.VectorSubcoreMesh(core_axis_name="c", subcore_axis_name="s")


@functools.partial(
    pl.kernel,
    mesh=_sc_mesh,
    out_type=jax.ShapeDtypeStruct((_SAMPLES, _BATCH, _HIDDEN), jnp.float32),
    scratch_types=[
        pltpu.VMEM((_SAMPLES, _MI), jnp.int32),
        pltpu.VMEM((_SAMPLES, _MI, _HIDDEN), jnp.float32),
        pltpu.SemaphoreType.DMA,
    ],
    compiler_params=pltpu.CompilerParams(
        use_tc_tiling_on_sc=False, needs_layout_passes=False
    ),
)
def _gather_sc(idx_hbm, table_hbm, out_hbm, idx_v, rows_v, sem):
    wid = lax.axis_index("s") * _NC + lax.axis_index("c")
    i0 = wid * _IPW

    def body(m, carry):
        r0 = i0 + m * _MI
        pltpu.sync_copy(idx_hbm.at[:, pl.ds(r0, _MI)], idx_v)
        copies = []
        for s in range(_SAMPLES):
            copies.append(
                pltpu.async_copy(
                    table_hbm.at[idx_v.at[s]], rows_v.at[s], sem
                )
            )
        for cp in copies:
            cp.wait()
        pltpu.sync_copy(rows_v, out_hbm.at[:, pl.ds(r0, _MI)])
        return carry

    lax.fori_loop(0, _NMACRO, body, 0)


def _mm_body(x_ref, bd_ref, b_ref, o_ref):
    acc = b_ref[...].astype(jnp.float32)
    for s in range(_SAMPLES):
        acc = acc + lax.dot_general(
            x_ref[s], bd_ref[s],
            (((1,), (0,)), ((), ())),
            preferred_element_type=jnp.float32,
        )
    o_ref[...] = acc


_BM4 = 512


def _project_tc(x3, BD, b128):
    return pl.pallas_call(
        _mm_body,
        grid=(_BATCH // _PACK // _BM4,),
        in_specs=[
            pl.BlockSpec((_SAMPLES, _BM4, 128), lambda i: (0, i, 0)),
            pl.BlockSpec((_SAMPLES, 128, 128), lambda i: (0, 0, 0)),
            pl.BlockSpec((1, 128), lambda i: (0, 0)),
        ],
        out_specs=pl.BlockSpec((_BM4, 128), lambda i: (i, 0)),
        out_shape=jax.ShapeDtypeStruct((_BATCH // _PACK, 128), jnp.float32),
    )(x3, BD, b128)


def kernel(all_values, table, W, b):
    idx_t = all_values.T
    emb = _gather_sc(idx_t, table)
    x3 = emb.reshape(_SAMPLES, _BATCH // _PACK, 128)
    WsT = W.reshape(_HIDDEN, _SAMPLES, _HIDDEN).transpose(1, 2, 0)
    eye4 = jnp.eye(_PACK, dtype=W.dtype)
    BD = jnp.einsum("de,sfh->sdfeh", eye4, WsT).reshape(_SAMPLES, 128, 128)
    b128 = jnp.tile(b, _PACK).reshape(1, 128)
    out = _project_tc(x3, BD, b128)
    return out.reshape(_BATCH, _HIDDEN)
